# Initial kernel scaffold; baseline (speedup 1.0000x reference)
import jax
import jax.numpy as jnp
from jax.experimental import pallas as pl


def kernel(obj_encs, n_nodes, W_g1, b_g1, a_g1, W_c1, b_c1, ln_g1, ln_b1, W_g2, b_g2, a_g2, W_c2, b_c2, ln_g2, ln_b2, W_g3, b_g3, a_g3, W_c3, b_c3, ln_g3, ln_b3, W_p, b_p):
    raise NotImplementedError("write your pallas kernel here")


# speedup vs baseline: 5.2000x; 5.2000x over previous
import functools

import jax
import jax.numpy as jnp
from jax import lax
from jax.experimental import pallas as pl
from jax.experimental.pallas import tpu as pltpu
from jax.experimental.pallas import tpu_sc as plsc

B, S, IN = 16, 512, 256
K = 10
KP = 16
YW = 128
N = B * S
L = 16
NC, NS = 2, 16


def _gat_hs_body(x_ref, wg, bg, ag, h_ref, s_ref):
    x = x_ref[...]
    h_ref[...] = jnp.dot(x, wg[...], preferred_element_type=jnp.float32) + bg[...]
    s_ref[...] = jnp.dot(x, ag[...], preferred_element_type=jnp.float32)


def _y_body(h_ref, al_ref, y_ref):
    h = h_ref[...]
    g = lax.dot_general(al_ref[...], h, (((0,), (0,)), ((), ())),
                        preferred_element_type=jnp.float32)
    y = h + g
    dh = y.shape[-1]
    if dh != YW:
        y = jnp.concatenate([y, jnp.zeros((S, YW - dh), jnp.float32)], axis=-1)
    y_ref[...] = y


def _d2_body(y_ref, sqc_ref, sqr_ref, d_ref):
    y = y_ref[...]
    gram = lax.dot_general(y, y, (((1,), (1,)), ((), ())),
                           preferred_element_type=jnp.float32)
    d_ref[...] = (sqc_ref[...] + sqr_ref[0]) - 2.0 * gram


def _pre_body(y_ref, gath, wc, bc, o_ref):
    dh = wc.shape[0] // 2
    yi = y_ref[...][:, :dh]
    yj = gath[...][:, :, :dh]
    edge = jnp.concatenate(
        [jnp.broadcast_to(yi[:, None, :], yj.shape), yj - yi[:, None, :]],
        axis=-1)
    m = lax.dot_general(edge, wc[...], (((2,), (0,)), ((), ())),
                        preferred_element_type=jnp.float32) + bc[...]
    o_ref[...] = jnp.max(m, axis=1)


def _final_body(x3_ref, x1_ref, x2_ref, wp1, wp2, wp3, bp, o_ref):
    o = (jnp.dot(x1_ref[...], wp1[...], preferred_element_type=jnp.float32)
         + jnp.dot(x2_ref[...], wp2[...], preferred_element_type=jnp.float32)
         + jnp.dot(x3_ref[...], wp3[...], preferred_element_type=jnp.float32)
         + bp[...])
    nrm = jnp.sqrt(jnp.sum(o * o, axis=-1, keepdims=True))
    o_ref[...] = o / (nrm + 1e-9)


def _seg_spec(d):
    return pl.BlockSpec((S, d), lambda i: (i, 0))


def _gath_spec():
    return pl.BlockSpec((S, KP, YW), lambda i: (i, 0, 0))


def _full_spec(shape):
    nd = len(shape)
    return pl.BlockSpec(shape, lambda i, _n=nd: (0,) * _n)


def _out(d):
    return jax.ShapeDtypeStruct((N, d), jnp.float32)


def _tc_gat_hs(x, wg, bg, ag, din, dh):
    return pl.pallas_call(
        _gat_hs_body, grid=(B,),
        in_specs=[_seg_spec(din)] + [_full_spec(w.shape) for w in (wg, bg, ag)],
        out_specs=[_seg_spec(dh), _seg_spec(1)],
        out_shape=[_out(dh), _out(1)],
    )(x, wg, bg, ag)


def _tc_y(h, al, dh):
    return pl.pallas_call(
        _y_body, grid=(B,),
        in_specs=[_seg_spec(dh), _seg_spec(1)],
        out_specs=_seg_spec(YW),
        out_shape=_out(YW),
    )(h, al)


def _tc_d2(y, sqc, sqr, dh):
    del dh
    return pl.pallas_call(
        _d2_body, grid=(B,),
        in_specs=[_seg_spec(YW), _seg_spec(1),
                  pl.BlockSpec((1, 1, S), lambda i: (i, 0, 0))],
        out_specs=_seg_spec(S),
        out_shape=_out(S),
    )(y, sqc, sqr)


def _tc_pre(y, gath, wc, bc, dout):
    return pl.pallas_call(
        _pre_body, grid=(B,),
        in_specs=[_seg_spec(YW), _gath_spec(),
                  _full_spec(wc.shape), _full_spec(bc.shape)],
        out_specs=_seg_spec(dout),
        out_shape=_out(dout),
    )(y, gath, wc, bc)


def _tc_final(x3, x1, x2, wp1, wp2, wp3, bp):
    return pl.pallas_call(
        _final_body, grid=(B,),
        in_specs=[_seg_spec(128), _seg_spec(128), _seg_spec(64)]
        + [_full_spec(w.shape) for w in (wp1, wp2, wp3, bp)],
        out_specs=pl.BlockSpec((S, 128), lambda i: (i, 0)),
        out_shape=jax.ShapeDtypeStruct((N, 128), jnp.float32),
    )(x3, x1, x2, wp1, wp2, wp3, bp)


def _sc_gather_nbrs(dsel, ypad):
    NW = NC * NS
    RPW = N // NW
    CH = 8
    mesh = plsc.VectorSubcoreMesh(core_axis_name="c", subcore_axis_name="s")

    @functools.partial(
        pl.kernel,
        out_type=jax.ShapeDtypeStruct((N * KP, YW), jnp.float32),
        mesh=mesh,
        compiler_params=pltpu.CompilerParams(needs_layout_passes=False),
        scratch_types=[
            pltpu.VMEM((CH, S), jnp.float32),
            pltpu.VMEM((CH * KP,), jnp.int32),
            pltpu.VMEM((CH * KP, YW), jnp.float32),
            pltpu.SemaphoreType.DMA,
        ],
    )
    def k(d_hbm, y_hbm, out_hbm, dv, iv, gv, sem):
        wid = lax.axis_index("s") * NC + lax.axis_index("c")
        base = wid * RPW
        segb = (base // S) * S
        lanes = lax.iota(jnp.int32, L)
        lane0 = lanes == 0

        def chunk(c, _):
            row0 = base + c * CH
            pltpu.sync_copy(d_hbm.at[pl.ds(row0, CH)], dv)

            def topk_row(r, _):
                def one_pass(p, carry):
                    m1 = jnp.full((L,), jnp.inf, jnp.float32)
                    i1 = jnp.zeros((L,), jnp.int32)
                    m2 = jnp.full((L,), jnp.inf, jnp.float32)
                    i2 = jnp.zeros((L,), jnp.int32)
                    g1 = lanes
                    g2 = lanes + L
                    for cc in range(0, S // L, 2):
                        v1 = dv[r, pl.ds(cc * L, L)]
                        lt1 = v1 < m1
                        m1 = jnp.where(lt1, v1, m1)
                        i1 = jnp.where(lt1, g1, i1)
                        g1 = g1 + 2 * L
                        v2 = dv[r, pl.ds((cc + 1) * L, L)]
                        lt2 = v2 < m2
                        m2 = jnp.where(lt2, v2, m2)
                        i2 = jnp.where(lt2, g2, i2)
                        g2 = g2 + 2 * L
                    lt = m2 < m1
                    m1 = jnp.where(lt, m2, m1)
                    i1 = jnp.where(lt, i2, i1)
                    _, si = plsc.sort_key_val(m1, i1)
                    plsc.store_scatter(
                        iv, [jnp.full((L,), r * KP + p, jnp.int32)],
                        si + segb, mask=lane0)
                    plsc.store_scatter(
                        iv, [jnp.full((L,), r * KP + K + p, jnp.int32)],
                        si + segb, mask=lane0 & (p < KP - K))
                    plsc.store_scatter(
                        dv, [jnp.full((L,), r, jnp.int32), si],
                        jnp.full((L,), jnp.inf, jnp.float32), mask=lane0)
                    return carry

                lax.fori_loop(0, K, one_pass, 0)
                return 0

            lax.fori_loop(0, CH, topk_row, 0)
            pltpu.async_copy(y_hbm.at[iv], gv, sem).wait()
            pltpu.sync_copy(gv, out_hbm.at[pl.ds(row0 * KP, CH * KP)])
            return 0

        lax.fori_loop(0, RPW // CH, chunk, 0)

    return k(dsel, ypad)


def kernel(obj_encs, n_nodes, W_g1, b_g1, a_g1, W_c1, b_c1, ln_g1, ln_b1,
           W_g2, b_g2, a_g2, W_c2, b_c2, ln_g2, ln_b2,
           W_g3, b_g3, a_g3, W_c3, b_c3, ln_g3, ln_b3, W_p, b_p):
    del n_nodes
    dims = [(IN, 128, 128), (128, 64, 64), (64, 32, 128)]
    layers = [
        (W_g1, b_g1, a_g1, W_c1, b_c1, ln_g1, ln_b1),
        (W_g2, b_g2, a_g2, W_c2, b_c2, ln_g2, ln_b2),
        (W_g3, b_g3, a_g3, W_c3, b_c3, ln_g3, ln_b3),
    ]

    def softmax_glue(s):
        return jax.nn.softmax(s.reshape(B, S), axis=1).reshape(N, 1)

    def sq_glue(y, dh):
        yr = y[:, :dh].reshape(B, S, dh)
        sq = jnp.sum(yr * yr, axis=-1)
        return sq.reshape(N, 1), sq.reshape(B, 1, S)

    def ln_selu_glue(pre, lg, lb, dout):
        z = pre.reshape(B, S, dout)
        mu = jnp.mean(z, axis=-1, keepdims=True)
        var = jnp.var(z, axis=-1, keepdims=True)
        z = (z - mu) / jnp.sqrt(var + 1e-5) * lg + lb
        return jax.nn.selu(z).reshape(N, dout)

    xs = []
    x = obj_encs
    for i in range(3):
        din, dh, dout = dims[i]
        wg, bg, ag, wc, bc, lg, lb = layers[i]
        h, s = _tc_gat_hs(x, wg, bg.reshape(1, -1), ag.reshape(-1, 1), din, dh)
        al = softmax_glue(s)
        y = _tc_y(h, al, dh)
        sqc, sqr = sq_glue(y, dh)
        d = _tc_d2(y, sqc, sqr, dh)
        gath = _sc_gather_nbrs(d, y).reshape(N, KP, YW)
        pre = _tc_pre(y, gath, wc, bc.reshape(1, -1), dout)
        x = ln_selu_glue(pre, lg, lb, dout)
        if i < 2:
            xs.append(x)

    out = _tc_final(x, xs[0], xs[1],
                    W_p[:128], W_p[128:192], W_p[192:],
                    b_p.reshape(1, -1))
    return out
